# baseline (speedup 1.0000x reference)
import functools

import jax
import jax.numpy as jnp
from jax import lax
from jax.experimental import pallas as pl
from jax.experimental.pallas import tpu as pltpu
from jax.experimental.pallas import tpu_sc as plsc

BATCH = 4
SEQ = 4096
D = 768
NW = 32
ROWS_PER_W = (BATCH * SEQ) // NW
CHUNK = 64
NCHUNK = ROWS_PER_W // CHUNK
DV = D // 16


def _emb_kernel(x_hbm, table_hbm, pos_hbm, out_hbm, idx_v, pos_v, rows_v, sem):
    cid = lax.axis_index("c")
    sid = lax.axis_index("s")
    wid = sid * 2 + cid
    base = wid * ROWS_PER_W
    s_start = lax.rem(base, SEQ)

    pltpu.sync_copy(x_hbm.at[pl.ds(base, ROWS_PER_W)], idx_v)

    def chunk_body(c, carry):
        r0 = c * CHUNK
        pltpu.sync_copy(pos_hbm.at[pl.ds(s_start + r0, CHUNK)], pos_v)
        pltpu.async_copy(table_hbm.at[idx_v.at[pl.ds(r0, CHUNK)]], rows_v,
                         sem).wait()

        def add_body(r, carry2):
            for j in range(DV):
                rows_v[r, pl.ds(j * 16, 16)] = (
                    rows_v[r, pl.ds(j * 16, 16)] + pos_v[r, pl.ds(j * 16, 16)])
            return carry2

        lax.fori_loop(0, CHUNK, add_body, 0)
        pltpu.sync_copy(rows_v, out_hbm.at[pl.ds(base + r0, CHUNK)])
        return carry

    lax.fori_loop(0, NCHUNK, chunk_body, 0)


@jax.jit
def kernel(x, table, pos_encoding):
    flat_idx = x.reshape(-1).astype(jnp.int32)
    mesh = plsc.VectorSubcoreMesh(core_axis_name="c", subcore_axis_name="s")
    run = functools.partial(
        pl.kernel,
        out_type=jax.ShapeDtypeStruct((BATCH * SEQ, D), jnp.float32),
        mesh=mesh,
        scratch_types=[
            pltpu.VMEM((ROWS_PER_W,), jnp.int32),
            pltpu.VMEM((CHUNK, D), jnp.float32),
            pltpu.VMEM((CHUNK, D), jnp.float32),
            pltpu.SemaphoreType.DMA,
        ],
    )(_emb_kernel)
    out = run(flat_idx, table, pos_encoding)
    return out.reshape(BATCH, SEQ, D)
